# baseline (speedup 1.0000x reference)
import functools
import jax
import jax.numpy as jnp
import numpy as np
from jax.experimental import pallas as pl
from jax.experimental.pallas import tpu as pltpu

_N = 10000
_E = 160000
_R = 16
_NB = 8
_D = 256
_H = 4
_HD = 64
_BN = 400
_GRID = _N // _BN


def _dense_body(s_ref, cnt_ref, x_ref, wstack_ref, root_ref, bias_ref,
                attf_ref, e1_ref, e1t_ref, hout_ref, alpha_ref):
    s = s_ref[...]
    cnt = cnt_ref[...]
    x = x_ref[...]
    inv = 1.0 / jnp.maximum(cnt, 1.0)
    m = s.reshape(_BN, _R, _D) * inv[:, :, None]
    m = m.reshape(_BN, _R * _D)
    agg = jnp.dot(m, wstack_ref[...], preferred_element_type=jnp.float32)
    agg = agg + jnp.dot(x, root_ref[...], preferred_element_type=jnp.float32)
    h = agg + bias_ref[...]
    p = h * attf_ref[...]
    score8 = jnp.dot(p, e1_ref[...], preferred_element_type=jnp.float32)
    score = score8[:, :_H]
    mx = jnp.max(score, axis=1, keepdims=True)
    ex = jnp.exp(score - mx)
    alpha = ex / jnp.sum(ex, axis=1, keepdims=True)
    e1t = e1t_ref[...][:_H]
    aexp = jnp.dot(alpha, e1t, preferred_element_type=jnp.float32)
    hout_ref[...] = h * aexp
    alpha_ref[...] = jnp.concatenate(
        [alpha, jnp.zeros((_BN, 8 - _H), jnp.float32)], axis=1)


def _dense_call(S2, cnt2, x, wstack, root, bias2, attf, e1, e1t):
    return pl.pallas_call(
        _dense_body,
        grid=(_GRID,),
        in_specs=[
            pl.BlockSpec((_BN, _R * _D), lambda i: (i, 0)),
            pl.BlockSpec((_BN, _R), lambda i: (i, 0)),
            pl.BlockSpec((_BN, _D), lambda i: (i, 0)),
            pl.BlockSpec((_R * _D, _D), lambda i: (0, 0)),
            pl.BlockSpec((_D, _D), lambda i: (0, 0)),
            pl.BlockSpec((1, _D), lambda i: (0, 0)),
            pl.BlockSpec((1, _D), lambda i: (0, 0)),
            pl.BlockSpec((_D, 8), lambda i: (0, 0)),
            pl.BlockSpec((8, _D), lambda i: (0, 0)),
        ],
        out_specs=[
            pl.BlockSpec((_BN, _D), lambda i: (i, 0)),
            pl.BlockSpec((_BN, 8), lambda i: (i, 0)),
        ],
        out_shape=[
            jax.ShapeDtypeStruct((_N, _D), jnp.float32),
            jax.ShapeDtypeStruct((_N, 8), jnp.float32),
        ],
    )(S2, cnt2, x, wstack, root, bias2, attf, e1, e1t)


def kernel(x, edge_index, edge_type, bases, comp, root, bias, att):
    src = edge_index[0]
    dst = edge_index[1]
    seg = dst * _R + edge_type
    S = jax.ops.segment_sum(x[src], seg, num_segments=_N * _R)
    cnt = jax.ops.segment_sum(jnp.ones((_E,), jnp.float32), seg,
                              num_segments=_N * _R)

    wstack = jnp.einsum('rb,bdo->rdo', comp, bases).reshape(_R * _D, _D)
    bias2 = bias.reshape(1, _D)
    attf = att.reshape(1, _H * _HD)
    head = np.repeat(np.arange(_H), _HD)
    e1 = np.zeros((_D, 8), np.float32)
    e1[np.arange(_D), head] = 1.0
    e1t = np.zeros((8, _D), np.float32)
    e1t[head, np.arange(_D)] = 1.0
    hout, alpha8 = _dense_call(S.reshape(_N, _R * _D), cnt.reshape(_N, _R),
                               x, wstack, root, bias2, attf,
                               jnp.asarray(e1), jnp.asarray(e1t))
    return hout, alpha8[:, :_H]
